# baseline (speedup 1.0000x reference)
import jax
import jax.numpy as jnp
from jax import lax
from jax.experimental import pallas as pl
from jax.experimental.pallas import tpu as pltpu
from jax.experimental.pallas import tpu_sc as plsc

_NC = 2
_NS = 16
_NW = _NC * _NS
_L = 16


def _sc_add(inp, tab, b_start, b_count, S, D):
    B = b_count
    seq_per_w = S // _NW
    C = 8
    n_chunks = seq_per_w // C
    n_triples = (n_chunks - 2) // 3

    mesh = plsc.VectorSubcoreMesh(core_axis_name="c", subcore_axis_name="s")

    @pl.kernel(
        out_type=jax.ShapeDtypeStruct((B, S, D), jnp.float32),
        mesh=mesh,
        scratch_types=[
            [pltpu.VMEM((C, D), jnp.float32) for _ in range(3)],
            [[pltpu.VMEM((C, D), jnp.float32) for _ in range(3)]
             for _ in range(B)],
            pltpu.SemaphoreType.DMA((3,)),
            pltpu.SemaphoreType.DMA((B, 3)),
            pltpu.SemaphoreType.DMA((B, 3)),
        ],
        compiler_params=pltpu.CompilerParams(use_tc_tiling_on_sc=True),
    )
    def body(inp_hbm, tab_hbm, out_hbm, tbufs, dbufs, tsems, lsems, ssems):
        w = lax.axis_index("s") * _NC + lax.axis_index("c")
        base = w * seq_per_w

        def tab_slice(c):
            return tab_hbm.at[pl.ds(base + c * C, C), :]

        def in_slice(c, b):
            return inp_hbm.at[b_start + b, pl.ds(base + c * C, C), :]

        def out_slice(c, b):
            return out_hbm.at[b, pl.ds(base + c * C, C), :]

        for p in range(3):
            pltpu.async_copy(tab_slice(p), tbufs[p], tsems.at[p])
        for p in range(2):
            for b in range(B):
                pltpu.async_copy(in_slice(p, b), dbufs[b][p], lsems.at[b, p])

        def step(c, par, store_wait_pred=None,
                 load_prefetch=True, tab_prefetch=True, tab_prefetch_pred=None):
            prv = (par + 2) % 3

            pltpu.make_async_copy(tab_slice(c), tbufs[par], tsems.at[par]).wait()
            for b in range(B):
                pltpu.make_async_copy(
                    in_slice(c, b), dbufs[b][par], lsems.at[b, par]
                ).wait()

            tbuf = tbufs[par]
            cur = [dbufs[b][par] for b in range(B)]

            @plsc.parallel_loop(0, D, _L)
            def add_body(i):
                sl = pl.ds(i, _L)
                for r in range(C):
                    t = tbuf[r, sl]
                    for b in range(B):
                        cur[b][r, sl] = cur[b][r, sl] + t

            for b in range(B):
                pltpu.async_copy(
                    dbufs[b][par], out_slice(c, b), ssems.at[b, par]
                )

            if load_prefetch:
                for b in range(B):
                    def _wait(b=b):
                        pltpu.make_async_copy(
                            dbufs[b][prv], out_slice(c - 1, b), ssems.at[b, prv]
                        ).wait()

                    if store_wait_pred is None:
                        _wait()
                    else:
                        pl.when(store_wait_pred)(_wait)
                for b in range(B):
                    pltpu.async_copy(
                        in_slice(c + 2, b), dbufs[b][prv], lsems.at[b, prv]
                    )

            if tab_prefetch:
                def _tpf():
                    pltpu.async_copy(tab_slice(c + 3), tbufs[par], tsems.at[par])

                if tab_prefetch_pred is None:
                    _tpf()
                else:
                    pl.when(tab_prefetch_pred)(_tpf)

        def triple_body(c3, carry):
            c0 = c3 * 3
            step(c0, 0, store_wait_pred=c3 > 0)
            step(c0 + 1, 1)
            step(c0 + 2, 2, tab_prefetch_pred=c3 < n_triples - 1)
            return carry

        lax.fori_loop(0, n_triples, triple_body, 0)

        step(n_chunks - 2, 0, load_prefetch=False, tab_prefetch=False)
        step(n_chunks - 1, 1, load_prefetch=False, tab_prefetch=False)

        for b in range(B):
            for p, c in ((2, n_chunks - 3), (0, n_chunks - 2), (1, n_chunks - 1)):
                pltpu.make_async_copy(
                    dbufs[b][p], out_slice(c, b), ssems.at[b, p]
                ).wait()

    return body(inp, tab)


def _tc_add(inp, tab, b_count, S, D):
    TS = 512

    def body(tab_ref, inp_ref, out_ref):
        out_ref[...] = inp_ref[...] + tab_ref[...]

    return pl.pallas_call(
        body,
        grid=(S // TS, b_count),
        in_specs=[
            pl.BlockSpec((TS, D), lambda s, b: (s, 0)),
            pl.BlockSpec((1, TS, D), lambda s, b: (b, s, 0)),
        ],
        out_specs=pl.BlockSpec((1, TS, D), lambda s, b: (b, s, 0)),
        out_shape=jax.ShapeDtypeStruct((b_count, S, D), jnp.float32),
    )(tab, inp)


def kernel(inp, embed_table):
    B, S, D = inp.shape
    tab = embed_table[:S]
    b_sc = B // 2
    sc_out = _sc_add(inp, tab, b_sc, B - b_sc, S, D)
    tc_out = _tc_add(inp, tab, b_sc, S, D)
    return jnp.concatenate([tc_out, sc_out], axis=0)


# speedup vs baseline: 1.6362x; 1.6362x over previous
import jax
import jax.numpy as jnp
from jax import lax
from jax.experimental import pallas as pl
from jax.experimental.pallas import tpu as pltpu
from jax.experimental.pallas import tpu_sc as plsc

_NC = 2
_NS = 16
_NW = _NC * _NS
_L = 16


def _sc_add(inp, tab, B, S, D):
    seq_per_w = S // _NW
    C = 8
    n_chunks = seq_per_w // C
    n_pairs = n_chunks // 2

    mesh = plsc.VectorSubcoreMesh(core_axis_name="c", subcore_axis_name="s")

    @pl.kernel(
        out_type=jax.ShapeDtypeStruct((B, S, D), jnp.float32),
        mesh=mesh,
        scratch_types=[
            [pltpu.VMEM((C, D), jnp.float32) for _ in range(2)],
            [[pltpu.VMEM((C, D), jnp.float32) for _ in range(2)]
             for _ in range(B)],
            pltpu.SemaphoreType.DMA((2,)),
            pltpu.SemaphoreType.DMA((B, 2)),
            pltpu.SemaphoreType.DMA((B, 2)),
        ],
        compiler_params=pltpu.CompilerParams(use_tc_tiling_on_sc=True),
    )
    def body(inp_hbm, tab_hbm, out_hbm, tbufs, dbufs, tsems, lsems, ssems):
        w = lax.axis_index("s") * _NC + lax.axis_index("c")
        base = w * seq_per_w

        def tab_slice(c):
            return tab_hbm.at[pl.ds(base + c * C, C), :]

        def in_slice(c, b):
            return inp_hbm.at[b, pl.ds(base + c * C, C), :]

        def out_slice(c, b):
            return out_hbm.at[b, pl.ds(base + c * C, C), :]

        pltpu.async_copy(tab_slice(0), tbufs[0], tsems.at[0])
        for b in range(B):
            pltpu.async_copy(in_slice(0, b), dbufs[b][0], lsems.at[b, 0])

        def step(c2, par):
            c = c2 * 2 + par
            nxt = 1 - par

            def tab_prefetch():
                pltpu.async_copy(tab_slice(c + 1), tbufs[nxt], tsems.at[nxt])

            if par == 0:
                tab_prefetch()
            else:
                pl.when(c2 < n_pairs - 1)(tab_prefetch)

            for b in range(B):
                def store_wait(b=b):
                    pltpu.make_async_copy(
                        dbufs[b][nxt], out_slice(c - 1, b), ssems.at[b, nxt]
                    ).wait()

                def inp_prefetch(b=b):
                    pltpu.async_copy(
                        in_slice(c + 1, b), dbufs[b][nxt], lsems.at[b, nxt]
                    )

                if par == 0:
                    pl.when(c2 > 0)(store_wait)
                    inp_prefetch()
                else:
                    store_wait()
                    pl.when(c2 < n_pairs - 1)(inp_prefetch)

            pltpu.make_async_copy(tab_slice(c), tbufs[par], tsems.at[par]).wait()
            for b in range(B):
                pltpu.make_async_copy(
                    in_slice(c, b), dbufs[b][par], lsems.at[b, par]
                ).wait()

            tbuf = tbufs[par]
            cur = [dbufs[b][par] for b in range(B)]

            for r in range(C):
                @plsc.parallel_loop(0, D, _L, unroll=4)
                def add_body(i):
                    sl = pl.ds(i, _L)
                    t = tbuf[r, sl]
                    for b in range(B):
                        cur[b][r, sl] = cur[b][r, sl] + t

            for b in range(B):
                pltpu.async_copy(
                    dbufs[b][par], out_slice(c, b), ssems.at[b, par]
                )

        def pair_body(c2, carry):
            step(c2, 0)
            step(c2, 1)
            return carry

        lax.fori_loop(0, n_pairs, pair_body, 0)

        for b in range(B):
            pltpu.make_async_copy(
                dbufs[b][1], out_slice(n_chunks - 1, b), ssems.at[b, 1]
            ).wait()

    return body(inp, tab)


def kernel(inp, embed_table):
    B, S, D = inp.shape
    return _sc_add(inp, embed_table[:S], B, S, D)


# speedup vs baseline: 1.6724x; 1.0221x over previous
import jax
import jax.numpy as jnp
from jax import lax
from jax.experimental import pallas as pl
from jax.experimental.pallas import tpu as pltpu
from jax.experimental.pallas import tpu_sc as plsc

_NC = 2
_NS = 16
_NW = _NC * _NS
_L = 16
_B_DIRECT = 2


def _sc_add(inp, tab, B, S, D):
    seq_per_w = S // _NW
    C = 8
    n_chunks = seq_per_w // C
    n_pairs = n_chunks // 2
    n_spmem = B - _B_DIRECT

    mesh = plsc.VectorSubcoreMesh(core_axis_name="c", subcore_axis_name="s")

    @pl.kernel(
        out_type=jax.ShapeDtypeStruct((B, S, D), jnp.float32),
        mesh=mesh,
        scratch_types=[
            [pltpu.VMEM((C, D), jnp.float32) for _ in range(2)],
            [[pltpu.VMEM((C, D), jnp.float32) for _ in range(2)]
             for _ in range(B)],
            pltpu.VMEM_SHARED((_NS, n_spmem, 2, C, D), jnp.float32),
            pltpu.SemaphoreType.DMA((2,)),
            pltpu.SemaphoreType.DMA((B, 2)),
            pltpu.SemaphoreType.DMA((B, 2)),
            pltpu.SemaphoreType.DMA((B, 2)),
        ],
        compiler_params=pltpu.CompilerParams(use_tc_tiling_on_sc=True),
    )
    def body(inp_hbm, tab_hbm, out_hbm, tbufs, dbufs, shared,
             tsems, lsems, xsems, ssems):
        w = lax.axis_index("s") * _NC + lax.axis_index("c")
        sid = lax.axis_index("s")
        base = w * seq_per_w

        def tab_slice(c):
            return tab_hbm.at[pl.ds(base + c * C, C), :]

        def in_slice(c, b):
            return inp_hbm.at[b, pl.ds(base + c * C, C), :]

        def out_slice(c, b):
            return out_hbm.at[b, pl.ds(base + c * C, C), :]

        pltpu.async_copy(tab_slice(0), tbufs[0], tsems.at[0])
        for b in range(B):
            pltpu.async_copy(in_slice(0, b), dbufs[b][0], lsems.at[b, 0])

        def step(c2, par):
            c = c2 * 2 + par
            nxt = 1 - par

            def tab_prefetch():
                pltpu.async_copy(tab_slice(c + 1), tbufs[nxt], tsems.at[nxt])

            if par == 0:
                tab_prefetch()
            else:
                pl.when(c2 < n_pairs - 1)(tab_prefetch)

            for b in range(B):
                if b < _B_DIRECT:
                    def drain(b=b):
                        pltpu.make_async_copy(
                            dbufs[b][nxt], out_slice(c - 1, b), ssems.at[b, nxt]
                        ).wait()
                else:
                    def drain(b=b):
                        pltpu.make_async_copy(
                            dbufs[b][nxt], shared.at[sid, b - _B_DIRECT, nxt],
                            xsems.at[b, nxt],
                        ).wait()
                        pltpu.async_copy(
                            shared.at[sid, b - _B_DIRECT, nxt],
                            out_slice(c - 1, b), ssems.at[b, nxt],
                        )

                def inp_prefetch(b=b):
                    pltpu.async_copy(
                        in_slice(c + 1, b), dbufs[b][nxt], lsems.at[b, nxt]
                    )

                if par == 0:
                    pl.when(c2 > 0)(drain)
                    inp_prefetch()
                else:
                    drain()
                    pl.when(c2 < n_pairs - 1)(inp_prefetch)

            pltpu.make_async_copy(tab_slice(c), tbufs[par], tsems.at[par]).wait()
            for b in range(B):
                pltpu.make_async_copy(
                    in_slice(c, b), dbufs[b][par], lsems.at[b, par]
                ).wait()

            tbuf = tbufs[par]
            cur = [dbufs[b][par] for b in range(B)]

            for r in range(C):
                @plsc.parallel_loop(0, D, _L, unroll=4)
                def add_body(i):
                    sl = pl.ds(i, _L)
                    t = tbuf[r, sl]
                    for b in range(B):
                        cur[b][r, sl] = cur[b][r, sl] + t

            for b in range(B):
                if b < _B_DIRECT:
                    pltpu.async_copy(
                        dbufs[b][par], out_slice(c, b), ssems.at[b, par]
                    )
                else:
                    def hop_wait(b=b):
                        pltpu.make_async_copy(
                            shared.at[sid, b - _B_DIRECT, par],
                            out_slice(c - 2, b), ssems.at[b, par],
                        ).wait()

                    pl.when(c2 > 0)(hop_wait)
                    pltpu.async_copy(
                        dbufs[b][par], shared.at[sid, b - _B_DIRECT, par],
                        xsems.at[b, par],
                    )

        def pair_body(c2, carry):
            step(c2, 0)
            step(c2, 1)
            return carry

        lax.fori_loop(0, n_pairs, pair_body, 0)

        for b in range(B):
            if b < _B_DIRECT:
                pltpu.make_async_copy(
                    dbufs[b][1], out_slice(n_chunks - 1, b), ssems.at[b, 1]
                ).wait()
            else:
                pltpu.make_async_copy(
                    dbufs[b][1], shared.at[sid, b - _B_DIRECT, 1], xsems.at[b, 1]
                ).wait()
                pltpu.async_copy(
                    shared.at[sid, b - _B_DIRECT, 1],
                    out_slice(n_chunks - 1, b), ssems.at[b, 1],
                )
        for b in range(_B_DIRECT, B):
            pltpu.make_async_copy(
                shared.at[sid, b - _B_DIRECT, 0],
                out_slice(n_chunks - 2, b), ssems.at[b, 0],
            ).wait()
            pltpu.make_async_copy(
                shared.at[sid, b - _B_DIRECT, 1],
                out_slice(n_chunks - 1, b), ssems.at[b, 1],
            ).wait()

    return body(inp, tab)


def kernel(inp, embed_table):
    B, S, D = inp.shape
    return _sc_add(inp, embed_table[:S], B, S, D)


# speedup vs baseline: 1.6819x; 1.0057x over previous
import jax
import jax.numpy as jnp
from jax import lax
from jax.experimental import pallas as pl
from jax.experimental.pallas import tpu as pltpu
from jax.experimental.pallas import tpu_sc as plsc

_NC = 2
_NS = 16
_NW = _NC * _NS
_L = 16
_B_DIRECT = 1


def _sc_add(inp, tab, B, S, D):
    seq_per_w = S // _NW
    C = 8
    n_chunks = seq_per_w // C
    n_pairs = n_chunks // 2
    n_spmem = B - _B_DIRECT

    mesh = plsc.VectorSubcoreMesh(core_axis_name="c", subcore_axis_name="s")

    @pl.kernel(
        out_type=jax.ShapeDtypeStruct((B, S, D), jnp.float32),
        mesh=mesh,
        scratch_types=[
            [pltpu.VMEM((C, D), jnp.float32) for _ in range(2)],
            [[pltpu.VMEM((C, D), jnp.float32) for _ in range(2)]
             for _ in range(B)],
            pltpu.VMEM_SHARED((_NS, n_spmem, 2, C, D), jnp.float32),
            pltpu.SemaphoreType.DMA((2,)),
            pltpu.SemaphoreType.DMA((B, 2)),
            pltpu.SemaphoreType.DMA((B, 2)),
            pltpu.SemaphoreType.DMA((B, 2)),
        ],
        compiler_params=pltpu.CompilerParams(use_tc_tiling_on_sc=True),
    )
    def body(inp_hbm, tab_hbm, out_hbm, tbufs, dbufs, shared,
             tsems, lsems, xsems, ssems):
        w = lax.axis_index("s") * _NC + lax.axis_index("c")
        sid = lax.axis_index("s")
        base = w * seq_per_w

        def tab_slice(c):
            return tab_hbm.at[pl.ds(base + c * C, C), :]

        def in_slice(c, b):
            return inp_hbm.at[b, pl.ds(base + c * C, C), :]

        def out_slice(c, b):
            return out_hbm.at[b, pl.ds(base + c * C, C), :]

        pltpu.async_copy(tab_slice(0), tbufs[0], tsems.at[0])
        for b in range(B):
            pltpu.async_copy(in_slice(0, b), dbufs[b][0], lsems.at[b, 0])

        def step(c2, par):
            c = c2 * 2 + par
            nxt = 1 - par

            def tab_prefetch():
                pltpu.async_copy(tab_slice(c + 1), tbufs[nxt], tsems.at[nxt])

            if par == 0:
                tab_prefetch()
            else:
                pl.when(c2 < n_pairs - 1)(tab_prefetch)

            for b in range(B):
                if b < _B_DIRECT:
                    def drain(b=b):
                        pltpu.make_async_copy(
                            dbufs[b][nxt], out_slice(c - 1, b), ssems.at[b, nxt]
                        ).wait()
                else:
                    def drain(b=b):
                        pltpu.make_async_copy(
                            dbufs[b][nxt], shared.at[sid, b - _B_DIRECT, nxt],
                            xsems.at[b, nxt],
                        ).wait()
                        pltpu.async_copy(
                            shared.at[sid, b - _B_DIRECT, nxt],
                            out_slice(c - 1, b), ssems.at[b, nxt],
                        )

                def inp_prefetch(b=b):
                    pltpu.async_copy(
                        in_slice(c + 1, b), dbufs[b][nxt], lsems.at[b, nxt]
                    )

                if par == 0:
                    pl.when(c2 > 0)(drain)
                    inp_prefetch()
                else:
                    drain()
                    pl.when(c2 < n_pairs - 1)(inp_prefetch)

            pltpu.make_async_copy(tab_slice(c), tbufs[par], tsems.at[par]).wait()
            for b in range(B):
                pltpu.make_async_copy(
                    in_slice(c, b), dbufs[b][par], lsems.at[b, par]
                ).wait()

            tbuf = tbufs[par]
            cur = [dbufs[b][par] for b in range(B)]

            for r in range(C):
                @plsc.parallel_loop(0, D, _L, unroll=4)
                def add_body(i):
                    sl = pl.ds(i, _L)
                    t = tbuf[r, sl]
                    for b in range(B):
                        cur[b][r, sl] = cur[b][r, sl] + t

            for b in range(B):
                if b < _B_DIRECT:
                    pltpu.async_copy(
                        dbufs[b][par], out_slice(c, b), ssems.at[b, par]
                    )
                else:
                    def hop_wait(b=b):
                        pltpu.make_async_copy(
                            shared.at[sid, b - _B_DIRECT, par],
                            out_slice(c - 2, b), ssems.at[b, par],
                        ).wait()

                    pl.when(c2 > 0)(hop_wait)
                    pltpu.async_copy(
                        dbufs[b][par], shared.at[sid, b - _B_DIRECT, par],
                        xsems.at[b, par],
                    )

        def pair_body(c2, carry):
            step(c2, 0)
            step(c2, 1)
            return carry

        lax.fori_loop(0, n_pairs, pair_body, 0)

        for b in range(B):
            if b < _B_DIRECT:
                pltpu.make_async_copy(
                    dbufs[b][1], out_slice(n_chunks - 1, b), ssems.at[b, 1]
                ).wait()
            else:
                pltpu.make_async_copy(
                    dbufs[b][1], shared.at[sid, b - _B_DIRECT, 1], xsems.at[b, 1]
                ).wait()
                pltpu.async_copy(
                    shared.at[sid, b - _B_DIRECT, 1],
                    out_slice(n_chunks - 1, b), ssems.at[b, 1],
                )
        for b in range(_B_DIRECT, B):
            pltpu.make_async_copy(
                shared.at[sid, b - _B_DIRECT, 0],
                out_slice(n_chunks - 2, b), ssems.at[b, 0],
            ).wait()
            pltpu.make_async_copy(
                shared.at[sid, b - _B_DIRECT, 1],
                out_slice(n_chunks - 1, b), ssems.at[b, 1],
            ).wait()

    return body(inp, tab)


def kernel(inp, embed_table):
    B, S, D = inp.shape
    return _sc_add(inp, embed_table[:S], B, S, D)
